# Initial kernel scaffold; baseline (speedup 1.0000x reference)
import jax
import jax.numpy as jnp
from jax.experimental import pallas as pl


def kernel(x, w_in, b_in, w_out, b_out):
    raise NotImplementedError("write your pallas kernel here")


# speedup vs baseline: 7.1629x; 7.1629x over previous
import functools

import jax
import jax.numpy as jnp
import numpy as np
from jax.experimental import pallas as pl
from jax.experimental.pallas import tpu as pltpu

_LEVELS = np.array([8, 5, 5, 5])
_G = 4
_NQ = 8
_L = 4
_DIM = 768
_DG = _DIM // _G
_GL = _G * _L
_TB = 1024


def _fsq_body(x_ref, w2t_ref, wout2_ref, bin_ref, bout_ref, scale_ref,
              bc_ref, idx_ref, q_ref, loss_ref, met_ref, hist_acc, loss_acc):
    i = pl.program_id(0)
    nsteps = pl.num_programs(0)

    @pl.when(i == 0)
    def _init():
        hist_acc[...] = jnp.zeros_like(hist_acc)
        loss_acc[0] = 0.0

    xblk = x_ref[...]
    z_tok = jax.lax.dot_general(
        xblk, w2t_ref[...], (((1,), (0,)), ((), ())),
        preferred_element_type=jnp.float32)
    z = z_tok.T + bin_ref[...]

    half_l = bc_ref[:, 0:1]
    offset = bc_ref[:, 1:2]
    shift = bc_ref[:, 2:3]
    half_w = bc_ref[:, 3:4]

    iota40 = jax.lax.broadcasted_iota(jnp.int32, (1, 40, 1), 1)
    iota25 = jax.lax.broadcasted_iota(jnp.int32, (1, 25, 1), 1)

    resid = z
    qout = jnp.zeros_like(z)
    hists = []
    for q in range(_NQ):
        scale = scale_ref[:, q:q + 1]
        zq = jnp.tanh(resid / scale + shift) * half_l - offset
        codes = jnp.round(zq)
        quant = (codes / half_w) * scale
        resid = resid - quant
        qout = qout + quant
        d = (codes + half_w).reshape(_G, _L, _TB)
        p = (d[:, 0, :] + 8.0 * d[:, 1, :]).astype(jnp.int32)
        h = (d[:, 2, :] + 5.0 * d[:, 3, :]).astype(jnp.int32)
        idx_ref[:, q, :] = p + 40 * h
        u = (p[:, None, :] == iota40).astype(jnp.float32)
        v = (h[:, None, :] == iota25).astype(jnp.float32)
        hq = jax.lax.dot_general(
            v, u, (((2,), (2,)), ((0,), (0,))),
            preferred_element_type=jnp.float32)
        hists.append(hq)
    hist_acc[...] += jnp.stack(hists, axis=1)

    out = jax.lax.dot_general(
        qout, wout2_ref[...], (((0,), (0,)), ((), ())),
        preferred_element_type=jnp.float32) + bout_ref[...]
    q_ref[...] = out
    diff = out - xblk
    loss_acc[0] += jnp.sum(diff * diff)

    @pl.when(i == nsteps - 1)
    def _fin():
        ntok = nsteps * _TB
        loss_ref[...] = jnp.full((1, 1), loss_acc[0] / float(ntok * _DIM))
        probs = hist_acc[...] * (1.0 / float(ntok))
        plogp = jnp.where(probs > 0, probs * jnp.log(probs + 1e-10), 0.0)
        ent = -jnp.sum(jnp.sum(plogp, axis=3), axis=2)
        met_ref[...] = jnp.exp(ent)


@jax.jit
def kernel(x, w_in, b_in, w_out, b_out):
    B, T, D = x.shape
    ntok = B * T
    nsteps = ntok // _TB
    xf = x.reshape(ntok, D)

    w2t = jax.scipy.linalg.block_diag(*[w_in[g] for g in range(_G)])
    wout2 = jax.scipy.linalg.block_diag(*[w_out[g] for g in range(_G)])
    bin_c = b_in.reshape(_GL, 1)
    bout_r = b_out.reshape(1, D)

    levels = jnp.tile(jnp.asarray(_LEVELS, jnp.float32), _G)
    eps = 1e-3
    half_l = (levels - 1.0) * (1.0 - eps) / 2.0
    offset = jnp.tile(jnp.where(jnp.asarray(_LEVELS % 2 == 0), 0.5, 0.0), _G)
    shift = jnp.arctanh(offset / half_l)
    half_w = jnp.tile(jnp.asarray(_LEVELS // 2, jnp.float32), _G)
    qs = jnp.arange(_NQ, dtype=jnp.float32)
    scales = (levels - 1.0)[:, None] ** (-qs[None, :])
    bconsts = jnp.stack([half_l, offset, shift, half_w], axis=1)

    const_spec = pl.BlockSpec(index_map=lambda i: (0, 0))
    idx_t, qf, loss, met = pl.pallas_call(
        _fsq_body,
        grid=(nsteps,),
        in_specs=[
            pl.BlockSpec((_TB, D), lambda i: (i, 0)),
            const_spec, const_spec, const_spec, const_spec, const_spec,
            const_spec,
        ],
        out_specs=[
            pl.BlockSpec((_G, _NQ, _TB), lambda i: (0, 0, i)),
            pl.BlockSpec((_TB, D), lambda i: (i, 0)),
            pl.BlockSpec((1, 1), lambda i: (0, 0)),
            pl.BlockSpec((_G, _NQ), lambda i: (0, 0)),
        ],
        out_shape=[
            jax.ShapeDtypeStruct((_G, _NQ, ntok), jnp.int32),
            jax.ShapeDtypeStruct((ntok, D), jnp.float32),
            jax.ShapeDtypeStruct((1, 1), jnp.float32),
            jax.ShapeDtypeStruct((_G, _NQ), jnp.float32),
        ],
        scratch_shapes=[
            pltpu.VMEM((_G, _NQ, 25, 40), jnp.float32),
            pltpu.SMEM((1,), jnp.float32),
        ],
        compiler_params=pltpu.CompilerParams(
            dimension_semantics=("arbitrary",)),
    )(xf, w2t, wout2, bin_c, bout_r, scales, bconsts)

    all_indices = idx_t.transpose(0, 2, 1).reshape(_G, B, T, _NQ)
    quantized = qf.reshape(B, T, D)
    return (all_indices, quantized, loss.reshape(()), met)
